# baseline (speedup 1.0000x reference)
import functools

import jax
import jax.numpy as jnp
from jax import lax
from jax.experimental import pallas as pl
from jax.experimental.pallas import tpu as pltpu
from jax.experimental.pallas import tpu_sc as plsc

BATCH_BLOCK = 64


def _gather_sc(emb_table, idx):
    B = idx.shape[0]
    _, D = emb_table.shape
    info = plsc.get_sparse_core_info()
    nw = info.num_cores * info.num_subcores
    b_per_w = B // nw
    mesh = plsc.VectorSubcoreMesh(core_axis_name="c", subcore_axis_name="s")

    @functools.partial(
        pl.kernel,
        mesh=mesh,
        out_type=jax.ShapeDtypeStruct((B, D), jnp.float32),
        scratch_types=[
            pltpu.VMEM((b_per_w,), jnp.int32),
            pltpu.VMEM((b_per_w, D), jnp.float32),
            pltpu.SemaphoreType.DMA,
        ],
        compiler_params=pltpu.CompilerParams(use_tc_tiling_on_sc=False),
    )
    def gather_kernel(table_hbm, idx_hbm, out_hbm, idx_v, rows_v, sem):
        wid = lax.axis_index("s") * info.num_cores + lax.axis_index("c")
        base = wid * b_per_w
        pltpu.sync_copy(idx_hbm.at[pl.ds(base, b_per_w)], idx_v)
        pltpu.async_copy(table_hbm.at[idx_v], rows_v, sem).wait()
        pltpu.sync_copy(rows_v, out_hbm.at[pl.ds(base, b_per_w)])

    return gather_kernel(emb_table, idx)


def _matmul_body(x_ref, wt_ref, b_ref, out_ref):
    acc = lax.dot_general(
        x_ref[...],
        wt_ref[...],
        (((1,), (0,)), ((), ())),
        preferred_element_type=jnp.float32,
    )
    out_ref[...] = acc + b_ref[...]


def _project(x, lin_wt, lin_b2d):
    B, D = x.shape
    V = lin_wt.shape[1]
    nb = pl.cdiv(B, BATCH_BLOCK)
    return pl.pallas_call(
        _matmul_body,
        grid=(nb,),
        in_specs=[
            pl.BlockSpec((BATCH_BLOCK, D), lambda j: (j, 0)),
            pl.BlockSpec((D, V), lambda j: (0, 0)),
            pl.BlockSpec((1, V), lambda j: (0, 0)),
        ],
        out_specs=pl.BlockSpec((BATCH_BLOCK, V), lambda j: (j, 0)),
        out_shape=jax.ShapeDtypeStruct((B, V), jnp.float32),
        compiler_params=pltpu.CompilerParams(
            vmem_limit_bytes=100 * 1024 * 1024,
        ),
    )(x, lin_wt, lin_b2d)


def kernel(inputs_, emb_table, lin_w, lin_b):
    idx = inputs_.astype(jnp.int32)
    x = _gather_sc(emb_table, idx)
    V = lin_w.shape[0]
    vpad = (V + 127) // 128 * 128
    wt = jnp.pad(lin_w.T, ((0, 0), (0, vpad - V)))
    b2 = jnp.pad(lin_b.reshape(1, -1), ((0, 0), (0, vpad - V)))
    return _project(x, wt, b2)[:, :V]


# speedup vs baseline: 1.0167x; 1.0167x over previous
import functools

import jax
import jax.numpy as jnp
from jax import lax
from jax.experimental import pallas as pl
from jax.experimental.pallas import tpu as pltpu
from jax.experimental.pallas import tpu_sc as plsc

BATCH_BLOCK = 32


def _gather_sc(emb_table, idx):
    B = idx.shape[0]
    _, D = emb_table.shape
    info = plsc.get_sparse_core_info()
    nw = info.num_cores * info.num_subcores
    b_per_w = B // nw
    mesh = plsc.VectorSubcoreMesh(core_axis_name="c", subcore_axis_name="s")

    @functools.partial(
        pl.kernel,
        mesh=mesh,
        out_type=jax.ShapeDtypeStruct((B, D), jnp.float32),
        scratch_types=[
            pltpu.VMEM((b_per_w,), jnp.int32),
            pltpu.VMEM((b_per_w, D), jnp.float32),
            pltpu.SemaphoreType.DMA,
        ],
        compiler_params=pltpu.CompilerParams(use_tc_tiling_on_sc=False),
    )
    def gather_kernel(table_hbm, idx_hbm, out_hbm, idx_v, rows_v, sem):
        wid = lax.axis_index("s") * info.num_cores + lax.axis_index("c")
        base = wid * b_per_w
        pltpu.sync_copy(idx_hbm.at[pl.ds(base, b_per_w)], idx_v)
        pltpu.async_copy(table_hbm.at[idx_v], rows_v, sem).wait()
        pltpu.sync_copy(rows_v, out_hbm.at[pl.ds(base, b_per_w)])

    return gather_kernel(emb_table, idx)


def _matmul_body(x_ref, wt_ref, b_ref, out_ref):
    acc = lax.dot_general(
        x_ref[...],
        wt_ref[...],
        (((1,), (0,)), ((), ())),
        preferred_element_type=jnp.float32,
    )
    out_ref[...] = acc + b_ref[...]


def _project(x, lin_wt, lin_b2d):
    B, D = x.shape
    V = lin_wt.shape[1]
    nb = pl.cdiv(B, BATCH_BLOCK)
    return pl.pallas_call(
        _matmul_body,
        grid=(nb,),
        in_specs=[
            pl.BlockSpec((BATCH_BLOCK, D), lambda j: (j, 0)),
            pl.BlockSpec((D, V), lambda j: (0, 0)),
            pl.BlockSpec((1, V), lambda j: (0, 0)),
        ],
        out_specs=pl.BlockSpec((BATCH_BLOCK, V), lambda j: (j, 0)),
        out_shape=jax.ShapeDtypeStruct((B, V), jnp.float32),
        compiler_params=pltpu.CompilerParams(
            vmem_limit_bytes=100 * 1024 * 1024,
        ),
    )(x, lin_wt, lin_b2d)


def kernel(inputs_, emb_table, lin_w, lin_b):
    idx = inputs_.astype(jnp.int32)
    x = _gather_sc(emb_table, idx)
    V = lin_w.shape[0]
    vpad = (V + 127) // 128 * 128
    wt = jnp.pad(lin_w.T, ((0, 0), (0, vpad - V)))
    b2 = jnp.pad(lin_b.reshape(1, -1), ((0, 0), (0, vpad - V)))
    return _project(x, wt, b2)[:, :V]
